# Initial kernel scaffold; baseline (speedup 1.0000x reference)
import jax
import jax.numpy as jnp
from jax.experimental import pallas as pl


def kernel(x, edge_index, edge_attr, batch, node_w, node_b, edge_w, edge_b, conv0_w1, conv0_b1, conv0_w2, conv0_b2, bn0_g, bn0_b, conv1_w1, conv1_b1, conv1_w2, conv1_b2, bn1_g, bn1_b, lin1_w, lin1_b, lin2_w, lin2_b):
    raise NotImplementedError("write your pallas kernel here")


# speedup vs baseline: 2.5296x; 2.5296x over previous
import functools

import jax
import jax.numpy as jnp
from jax import lax
from jax.experimental import pallas as pl
from jax.experimental.pallas import tpu as pltpu
from jax.experimental.pallas import tpu_sc as plsc

NC = 2
NS = 16
DH = 16
EPS_BN_ = 1e-5


def _chunk(rows, cap):
    ch = min(rows, cap)
    while rows % ch:
        ch -= 1
    return ch


def _chunk8(total, cap):
    cand = [d for d in range(8, cap + 1, 8) if total % d == 0]
    assert cand, (total, cap)
    good = [d for d in cand if total // d >= NS]
    return max(good) if good else min(cand)


def _edge_phase(hs, src, dst, attr_flat, ew2, eb2, *, interpret=False):
    _, N, _ = hs.shape
    E = src.shape[0]
    ew = E // NS
    W = _chunk(ew, 1000)
    nwin = ew // W
    ch = _chunk8(N, min(1000, W))
    nq = N // ch

    mesh = plsc.VectorSubcoreMesh(core_axis_name="c", subcore_axis_name="s",
                                  num_cores=NC, num_subcores=NS)

    def body(hs_hbm, src_hbm, dst_hbm, attr_hbm, ew_hbm, eb_hbm,
             out0, out1, ew_v, eb_v, idx_s, idx_d, att_v, row_v, aggr_sh):
        c = lax.axis_index("c")
        s = lax.axis_index("s")

        zero16 = jnp.zeros((DH,), jnp.float32)

        def zrow(i, carry):
            row_v[i] = zero16
            return carry
        lax.fori_loop(0, ch, zrow, 0)
        nq_s = lax.div(jnp.int32(nq) - s + jnp.int32(NS) - 1, jnp.int32(NS))

        def zchunk(k, carry):
            off = pl.multiple_of((s + k * NS) * ch, 8)
            pltpu.sync_copy(row_v.at[pl.ds(0, ch)],
                            aggr_sh.at[pl.ds(off, ch)])
            return carry
        lax.fori_loop(0, nq_s, zchunk, 0)
        plsc.subcore_barrier()

        for cc in range(NC):
            @pl.when(c == cc)
            def _():
                pltpu.sync_copy(ew_hbm.at[cc], ew_v)
                pltpu.sync_copy(eb_hbm.at[cc], eb_v)
        w0 = ew_v[pl.ds(0, DH)]
        w1 = ew_v[pl.ds(DH, DH)]
        w2 = ew_v[pl.ds(2 * DH, DH)]
        bb = eb_v[...]

        base0 = s * ew

        def window(g, carry):
            base = pl.multiple_of(base0 + g * W, 8)
            base3 = pl.multiple_of(base * 3, 8)
            pltpu.sync_copy(src_hbm.at[pl.ds(base, W)], idx_s)
            pltpu.sync_copy(dst_hbm.at[pl.ds(base, W)], idx_d)
            pltpu.sync_copy(attr_hbm.at[pl.ds(base3, 3 * W)],
                            att_v.at[pl.ds(0, 3 * W)])
            for cc in range(NC):
                @pl.when(c == cc)
                def _():
                    pltpu.sync_copy(hs_hbm.at[cc].at[idx_s], row_v)

            def edge(j, icarry):
                av = att_v[pl.ds(j * 3, DH)]
                hv = row_v[j]
                m = jnp.maximum(
                    hv + bb + av[0] * w0 + av[1] * w1 + av[2] * w2, 0.0)
                row_v[j] = m
                return icarry
            lax.fori_loop(0, W, edge, 0, unroll=4)
            pltpu.sync_copy(row_v, aggr_sh.at[idx_d], add=True)
            return carry
        lax.fori_loop(0, nwin, window, 0)
        plsc.subcore_barrier()

        def fchunk(k, carry):
            off = pl.multiple_of((s + k * NS) * ch, 8)
            sl = pl.ds(off, ch)
            pltpu.sync_copy(aggr_sh.at[sl], row_v.at[pl.ds(0, ch)])
            @pl.when(c == 0)
            def _():
                pltpu.sync_copy(row_v.at[pl.ds(0, ch)], out0.at[sl])
            @pl.when(c == 1)
            def _():
                pltpu.sync_copy(row_v.at[pl.ds(0, ch)], out1.at[sl])
            return carry
        lax.fori_loop(0, nq_s, fchunk, 0)

    f = pl.kernel(
        body,
        out_type=(jax.ShapeDtypeStruct((N, DH), jnp.float32),
                  jax.ShapeDtypeStruct((N, DH), jnp.float32)),
        mesh=mesh,
        scratch_types=[
            pltpu.VMEM((3 * DH,), jnp.float32),
            pltpu.VMEM((DH,), jnp.float32),
            pltpu.VMEM((W,), jnp.int32),
            pltpu.VMEM((W,), jnp.int32),
            pltpu.VMEM((3 * W + DH,), jnp.float32),
            pltpu.VMEM((W, DH), jnp.float32),
            pltpu.VMEM_SHARED((N, DH), jnp.float32),
        ],
        compiler_params=pltpu.CompilerParams(use_tc_tiling_on_sc=False),
        interpret=interpret,
    )
    return f(hs, src, dst, attr_flat, ew2, eb2)


def _pool_phase(hs, batch, G, *, interpret=False):
    _, N, _ = hs.shape
    W = _chunk(N, 1000)
    nwin_total = N // W
    ch = _chunk8(G, min(1000, W))
    nq = G // ch

    mesh = plsc.VectorSubcoreMesh(core_axis_name="c", subcore_axis_name="s",
                                  num_cores=NC, num_subcores=NS)

    def body(hs_hbm, b_hbm, p0, p1, cnt, idx_b, row_v, one_v, psum_sh, cnt_sh):
        c = lax.axis_index("c")
        s = lax.axis_index("s")

        zero16 = jnp.zeros((DH,), jnp.float32)
        one16 = jnp.ones((DH,), jnp.float32)

        def fill(i, carry):
            row_v[i] = zero16
            one_v[i] = one16
            return carry
        lax.fori_loop(0, W, fill, 0)
        nq_s = lax.div(jnp.int32(nq) - s + jnp.int32(NS) - 1, jnp.int32(NS))

        def zchunk(k, carry):
            sl = pl.ds(pl.multiple_of((s + k * NS) * ch, 8), ch)
            pltpu.sync_copy(row_v.at[pl.ds(0, ch)], psum_sh.at[sl])
            @pl.when(c == 0)
            def _():
                pltpu.sync_copy(row_v.at[pl.ds(0, ch)], cnt_sh.at[sl])
            return carry
        lax.fori_loop(0, nq_s, zchunk, 0)
        plsc.subcore_barrier()

        nw = lax.div(jnp.int32(nwin_total) - s + jnp.int32(NS) - 1, jnp.int32(NS))

        def window(k, carry):
            wid = s + k * NS
            base = pl.multiple_of(wid * W, 8)
            pltpu.sync_copy(b_hbm.at[pl.ds(base, W)], idx_b)
            for cc in range(NC):
                @pl.when(c == cc)
                def _():
                    pltpu.sync_copy(hs_hbm.at[cc].at[pl.ds(base, W)], row_v)
            pltpu.sync_copy(row_v, psum_sh.at[idx_b], add=True)
            @pl.when(c == 0)
            def _():
                pltpu.sync_copy(one_v, cnt_sh.at[idx_b], add=True)
            return carry
        lax.fori_loop(0, nw, window, 0)
        plsc.subcore_barrier()

        def fchunk(k, carry):
            sl = pl.ds(pl.multiple_of((s + k * NS) * ch, 8), ch)
            pltpu.sync_copy(psum_sh.at[sl], row_v.at[pl.ds(0, ch)])
            @pl.when(c == 0)
            def _():
                pltpu.sync_copy(row_v.at[pl.ds(0, ch)], p0.at[sl])
                pltpu.sync_copy(cnt_sh.at[sl], one_v.at[pl.ds(0, ch)])
                pltpu.sync_copy(one_v.at[pl.ds(0, ch)], cnt.at[sl])
            @pl.when(c == 1)
            def _():
                pltpu.sync_copy(row_v.at[pl.ds(0, ch)], p1.at[sl])
            return carry
        lax.fori_loop(0, nq_s, fchunk, 0)

    f = pl.kernel(
        body,
        out_type=(jax.ShapeDtypeStruct((G, DH), jnp.float32),
                  jax.ShapeDtypeStruct((G, DH), jnp.float32),
                  jax.ShapeDtypeStruct((G, DH), jnp.float32)),
        mesh=mesh,
        scratch_types=[
            pltpu.VMEM((W,), jnp.int32),
            pltpu.VMEM((W, DH), jnp.float32),
            pltpu.VMEM((W, DH), jnp.float32),
            pltpu.VMEM_SHARED((G, DH), jnp.float32),
            pltpu.VMEM_SHARED((G, DH), jnp.float32),
        ],
        compiler_params=pltpu.CompilerParams(use_tc_tiling_on_sc=False),
        interpret=interpret,
    )
    return f(hs, batch)


def _embed_tc(x, node_w, node_b, *, interpret=False):
    N, F = x.shape
    D = node_w.shape[1]
    B = _chunk(N, 10000)

    def body(x_ref, w_ref, b_ref, o_ref):
        o_ref[...] = jnp.dot(x_ref[...], w_ref[...],
                             preferred_element_type=jnp.float32) + b_ref[...]

    return pl.pallas_call(
        body,
        grid=(N // B,),
        in_specs=[pl.BlockSpec((B, F), lambda i: (i, 0)),
                  pl.BlockSpec((F, D), lambda i: (0, 0)),
                  pl.BlockSpec((1, D), lambda i: (0, 0))],
        out_specs=pl.BlockSpec((B, D), lambda i: (i, 0)),
        out_shape=jax.ShapeDtypeStruct((N, D), jnp.float32),
        interpret=interpret,
    )(x, node_w, node_b.reshape(1, D))


def _mlp_stats_tc(h, a0, a1, w1, b1, w2, b2, *, interpret=False):
    N, D = h.shape
    H = w1.shape[1]
    B = _chunk(N, 10000)

    def body(h_ref, a0_ref, a1_ref, w1_ref, b1_ref, w2_ref, b2_ref,
             z_ref, s_ref, q_ref):
        i = pl.program_id(0)
        xx = h_ref[...] + jnp.concatenate([a0_ref[...], a1_ref[...]], axis=1)
        t = jnp.maximum(jnp.dot(xx, w1_ref[...],
                                preferred_element_type=jnp.float32)
                        + b1_ref[...], 0.0)
        z = jnp.dot(t, w2_ref[...], preferred_element_type=jnp.float32) \
            + b2_ref[...]
        z_ref[...] = z

        @pl.when(i == 0)
        def _():
            s_ref[...] = jnp.zeros_like(s_ref)
            q_ref[...] = jnp.zeros_like(q_ref)
        s_ref[...] += jnp.sum(z, axis=0, keepdims=True)
        q_ref[...] += jnp.sum(z * z, axis=0, keepdims=True)

    return pl.pallas_call(
        body,
        grid=(N // B,),
        in_specs=[pl.BlockSpec((B, D), lambda i: (i, 0)),
                  pl.BlockSpec((B, DH), lambda i: (i, 0)),
                  pl.BlockSpec((B, DH), lambda i: (i, 0)),
                  pl.BlockSpec((D, H), lambda i: (0, 0)),
                  pl.BlockSpec((1, H), lambda i: (0, 0)),
                  pl.BlockSpec((H, D), lambda i: (0, 0)),
                  pl.BlockSpec((1, D), lambda i: (0, 0))],
        out_specs=(pl.BlockSpec((B, D), lambda i: (i, 0)),
                   pl.BlockSpec((1, D), lambda i: (0, 0)),
                   pl.BlockSpec((1, D), lambda i: (0, 0))),
        out_shape=(jax.ShapeDtypeStruct((N, D), jnp.float32),
                   jax.ShapeDtypeStruct((1, D), jnp.float32),
                   jax.ShapeDtypeStruct((1, D), jnp.float32)),
        interpret=interpret,
    )(h, a0, a1, w1, b1.reshape(1, H), w2, b2.reshape(1, D))


def _bn_relu_tc(z, zsum, zsq, g, bt, n_rows, *, interpret=False):
    N, D = z.shape
    B = _chunk(N, 10000)

    def body(z_ref, s_ref, q_ref, g_ref, b_ref, o_ref):
        inv_n = jnp.float32(1.0 / n_rows)
        mean = s_ref[...] * inv_n
        var = q_ref[...] * inv_n - mean * mean
        scale = g_ref[...] * lax.rsqrt(var + EPS_BN_)
        shift = b_ref[...] - mean * scale
        o_ref[...] = jnp.maximum(z_ref[...] * scale + shift, 0.0)

    return pl.pallas_call(
        body,
        grid=(N // B,),
        in_specs=[pl.BlockSpec((B, D), lambda i: (i, 0)),
                  pl.BlockSpec((1, D), lambda i: (0, 0)),
                  pl.BlockSpec((1, D), lambda i: (0, 0)),
                  pl.BlockSpec((1, D), lambda i: (0, 0)),
                  pl.BlockSpec((1, D), lambda i: (0, 0))],
        out_specs=pl.BlockSpec((B, D), lambda i: (i, 0)),
        out_shape=jax.ShapeDtypeStruct((N, D), jnp.float32),
        interpret=interpret,
    )(z, zsum, zsq, g.reshape(1, D), bt.reshape(1, D))


def _head_tc(p0, p1, cnt, l1w, l1b, l2w, l2b, *, interpret=False):
    G = p0.shape[0]
    D = 2 * DH
    H = l1w.shape[1]
    O = l2w.shape[1]

    def body(p0_ref, p1_ref, c_ref, w1_ref, b1_ref, w2_ref, b2_ref, o_ref):
        ssum = jnp.concatenate([p0_ref[...], p1_ref[...]], axis=1)
        c = jnp.maximum(c_ref[...][:, 0:1], 1.0)
        gx = ssum / c
        t = jnp.maximum(jnp.dot(gx, w1_ref[...],
                                preferred_element_type=jnp.float32)
                        + b1_ref[...], 0.0)
        o_ref[...] = jnp.dot(t, w2_ref[...],
                             preferred_element_type=jnp.float32) + b2_ref[...]

    return pl.pallas_call(
        body,
        out_shape=jax.ShapeDtypeStruct((G, O), jnp.float32),
        interpret=interpret,
    )(p0, p1, cnt, l1w, l1b.reshape(1, H), l2w, l2b.reshape(1, O))


def _split(h):
    n, d = h.shape
    return jnp.transpose(h.reshape(n, 2, DH), (1, 0, 2))


def _forward(x, edge_index, edge_attr, batch,
             node_w, node_b, edge_w, edge_b,
             conv0_w1, conv0_b1, conv0_w2, conv0_b2, bn0_g, bn0_b,
             conv1_w1, conv1_b1, conv1_w2, conv1_b2, bn1_g, bn1_b,
             lin1_w, lin1_b, lin2_w, lin2_b, G, interpret=False):
    N = x.shape[0]
    src = edge_index[0]
    dst = edge_index[1]
    ew2 = edge_w.reshape(3, 2, DH).transpose(1, 0, 2).reshape(2, 3 * DH)
    eb2 = edge_b.reshape(2, DH)
    attr_flat = edge_attr.reshape(-1)

    h = _embed_tc(x, node_w, node_b, interpret=interpret)
    layers = [
        (conv0_w1, conv0_b1, conv0_w2, conv0_b2, bn0_g, bn0_b),
        (conv1_w1, conv1_b1, conv1_w2, conv1_b2, bn1_g, bn1_b),
    ]
    for (w1, b1, w2, b2, g, bt) in layers:
        hs = _split(h)
        a0, a1 = _edge_phase(hs, src, dst, attr_flat, ew2, eb2,
                             interpret=interpret)
        z, zsum, zsq = _mlp_stats_tc(h, a0, a1, w1, b1, w2, b2,
                                     interpret=interpret)
        h = _bn_relu_tc(z, zsum, zsq, g, bt, N, interpret=interpret)

    hs = _split(h)
    p0, p1, cnt = _pool_phase(hs, batch, G, interpret=interpret)
    return _head_tc(p0, p1, cnt, lin1_w, lin1_b, lin2_w, lin2_b,
                    interpret=interpret)


def kernel(x, edge_index, edge_attr, batch,
           node_w, node_b, edge_w, edge_b,
           conv0_w1, conv0_b1, conv0_w2, conv0_b2, bn0_g, bn0_b,
           conv1_w1, conv1_b1, conv1_w2, conv1_b2, bn1_g, bn1_b,
           lin1_w, lin1_b, lin2_w, lin2_b):
    return _forward(x, edge_index, edge_attr, batch,
                    node_w, node_b, edge_w, edge_b,
                    conv0_w1, conv0_b1, conv0_w2, conv0_b2, bn0_g, bn0_b,
                    conv1_w1, conv1_b1, conv1_w2, conv1_b2, bn1_g, bn1_b,
                    lin1_w, lin1_b, lin2_w, lin2_b, G=2000)


# speedup vs baseline: 2.5451x; 1.0061x over previous
import functools

import jax
import jax.numpy as jnp
from jax import lax
from jax.experimental import pallas as pl
from jax.experimental.pallas import tpu as pltpu
from jax.experimental.pallas import tpu_sc as plsc

NC = 2
NS = 16
DH = 16
EPS_BN_ = 1e-5


def _chunk(rows, cap):
    ch = min(rows, cap)
    while rows % ch:
        ch -= 1
    return ch


def _chunk8(total, cap):
    cand = [d for d in range(8, cap + 1, 8) if total % d == 0]
    assert cand, (total, cap)
    good = [d for d in cand if total // d >= NS]
    return max(good) if good else min(cand)


def _edge_phase(h0t, h1t, src, dst, attr_flat, ew2, eb2, *, interpret=False):
    N = h0t.shape[0]
    E = src.shape[0]
    ew = E // NS
    W = _chunk(ew, 1000)
    nwin = ew // W
    ch = _chunk8(N, min(1000, W))
    nq = N // ch

    mesh = plsc.VectorSubcoreMesh(core_axis_name="c", subcore_axis_name="s",
                                  num_cores=NC, num_subcores=NS)

    def body(h0_hbm, h1_hbm, src_hbm, dst_hbm, attr_hbm, ew_hbm, eb_hbm,
             out0, out1, ew_v, eb_v, idx_s, idx_d, att_v, row_v, aggr_sh):
        c = lax.axis_index("c")
        s = lax.axis_index("s")

        zero16 = jnp.zeros((DH,), jnp.float32)

        def zrow(i, carry):
            row_v[i] = zero16
            return carry
        lax.fori_loop(0, ch, zrow, 0)
        nq_s = lax.div(jnp.int32(nq) - s + jnp.int32(NS) - 1, jnp.int32(NS))

        def zchunk(k, carry):
            off = pl.multiple_of((s + k * NS) * ch, 8)
            pltpu.sync_copy(row_v.at[pl.ds(0, ch)],
                            aggr_sh.at[pl.ds(off, ch)])
            return carry
        lax.fori_loop(0, nq_s, zchunk, 0)
        plsc.subcore_barrier()

        for cc in range(NC):
            @pl.when(c == cc)
            def _():
                pltpu.sync_copy(ew_hbm.at[pl.ds(cc * 3 * DH, 3 * DH)], ew_v)
                pltpu.sync_copy(eb_hbm.at[pl.ds(cc * DH, DH)], eb_v)
        w0 = ew_v[pl.ds(0, DH)]
        w1 = ew_v[pl.ds(DH, DH)]
        w2 = ew_v[pl.ds(2 * DH, DH)]
        bb = eb_v[...]

        base0 = s * ew

        def window(g, carry):
            base = pl.multiple_of(base0 + g * W, 8)
            base3 = pl.multiple_of(base * 3, 8)
            pltpu.sync_copy(src_hbm.at[pl.ds(base, W)], idx_s)
            pltpu.sync_copy(dst_hbm.at[pl.ds(base, W)], idx_d)
            pltpu.sync_copy(attr_hbm.at[pl.ds(base3, 3 * W)],
                            att_v.at[pl.ds(0, 3 * W)])
            @pl.when(c == 0)
            def _():
                pltpu.sync_copy(h0_hbm.at[idx_s], row_v)
            @pl.when(c == 1)
            def _():
                pltpu.sync_copy(h1_hbm.at[idx_s], row_v)

            def edge(j, icarry):
                av = att_v[pl.ds(j * 3, DH)]
                hv = row_v[j]
                m = jnp.maximum(
                    hv + bb + av[0] * w0 + av[1] * w1 + av[2] * w2, 0.0)
                row_v[j] = m
                return icarry
            lax.fori_loop(0, W, edge, 0, unroll=4)
            pltpu.sync_copy(row_v, aggr_sh.at[idx_d], add=True)
            return carry
        lax.fori_loop(0, nwin, window, 0)
        plsc.subcore_barrier()

        def fchunk(k, carry):
            off = pl.multiple_of((s + k * NS) * ch, 8)
            sl = pl.ds(off, ch)
            pltpu.sync_copy(aggr_sh.at[sl], row_v.at[pl.ds(0, ch)])
            @pl.when(c == 0)
            def _():
                pltpu.sync_copy(row_v.at[pl.ds(0, ch)], out0.at[sl])
            @pl.when(c == 1)
            def _():
                pltpu.sync_copy(row_v.at[pl.ds(0, ch)], out1.at[sl])
            return carry
        lax.fori_loop(0, nq_s, fchunk, 0)

    f = pl.kernel(
        body,
        out_type=(jax.ShapeDtypeStruct((N, DH), jnp.float32),
                  jax.ShapeDtypeStruct((N, DH), jnp.float32)),
        mesh=mesh,
        scratch_types=[
            pltpu.VMEM((3 * DH,), jnp.float32),
            pltpu.VMEM((DH,), jnp.float32),
            pltpu.VMEM((W,), jnp.int32),
            pltpu.VMEM((W,), jnp.int32),
            pltpu.VMEM((3 * W + DH,), jnp.float32),
            pltpu.VMEM((W, DH), jnp.float32),
            pltpu.VMEM_SHARED((N, DH), jnp.float32),
        ],
        compiler_params=pltpu.CompilerParams(use_tc_tiling_on_sc=False),
        interpret=interpret,
    )
    return f(h0t, h1t, src, dst, attr_flat, ew2, eb2)


def _pool_phase(h0t, h1t, batch, G, *, interpret=False):
    N = h0t.shape[0]
    W = _chunk(N, 1000)
    nwin_total = N // W
    ch = _chunk8(G, min(1000, W))
    nq = G // ch

    mesh = plsc.VectorSubcoreMesh(core_axis_name="c", subcore_axis_name="s",
                                  num_cores=NC, num_subcores=NS)

    def body(h0_hbm, h1_hbm, b_hbm, p0, p1, cnt, idx_b, row_v, one_v,
             psum_sh, cnt_sh):
        c = lax.axis_index("c")
        s = lax.axis_index("s")

        zero16 = jnp.zeros((DH,), jnp.float32)
        one16 = jnp.ones((DH,), jnp.float32)

        def fill(i, carry):
            row_v[i] = zero16
            one_v[i] = one16
            return carry
        lax.fori_loop(0, W, fill, 0)
        nq_s = lax.div(jnp.int32(nq) - s + jnp.int32(NS) - 1, jnp.int32(NS))

        def zchunk(k, carry):
            sl = pl.ds(pl.multiple_of((s + k * NS) * ch, 8), ch)
            pltpu.sync_copy(row_v.at[pl.ds(0, ch)], psum_sh.at[sl])
            @pl.when(c == 0)
            def _():
                pltpu.sync_copy(row_v.at[pl.ds(0, ch)], cnt_sh.at[sl])
            return carry
        lax.fori_loop(0, nq_s, zchunk, 0)
        plsc.subcore_barrier()

        nw = lax.div(jnp.int32(nwin_total) - s + jnp.int32(NS) - 1, jnp.int32(NS))

        def window(k, carry):
            wid = s + k * NS
            base = pl.multiple_of(wid * W, 8)
            pltpu.sync_copy(b_hbm.at[pl.ds(base, W)], idx_b)
            @pl.when(c == 0)
            def _():
                pltpu.sync_copy(h0_hbm.at[pl.ds(base, W)], row_v)
            @pl.when(c == 1)
            def _():
                pltpu.sync_copy(h1_hbm.at[pl.ds(base, W)], row_v)
            pltpu.sync_copy(row_v, psum_sh.at[idx_b], add=True)
            @pl.when(c == 0)
            def _():
                pltpu.sync_copy(one_v, cnt_sh.at[idx_b], add=True)
            return carry
        lax.fori_loop(0, nw, window, 0)
        plsc.subcore_barrier()

        def fchunk(k, carry):
            sl = pl.ds(pl.multiple_of((s + k * NS) * ch, 8), ch)
            pltpu.sync_copy(psum_sh.at[sl], row_v.at[pl.ds(0, ch)])
            @pl.when(c == 0)
            def _():
                pltpu.sync_copy(row_v.at[pl.ds(0, ch)], p0.at[sl])
                pltpu.sync_copy(cnt_sh.at[sl], one_v.at[pl.ds(0, ch)])
                pltpu.sync_copy(one_v.at[pl.ds(0, ch)], cnt.at[sl])
            @pl.when(c == 1)
            def _():
                pltpu.sync_copy(row_v.at[pl.ds(0, ch)], p1.at[sl])
            return carry
        lax.fori_loop(0, nq_s, fchunk, 0)

    f = pl.kernel(
        body,
        out_type=(jax.ShapeDtypeStruct((G, DH), jnp.float32),
                  jax.ShapeDtypeStruct((G, DH), jnp.float32),
                  jax.ShapeDtypeStruct((G, DH), jnp.float32)),
        mesh=mesh,
        scratch_types=[
            pltpu.VMEM((W,), jnp.int32),
            pltpu.VMEM((W, DH), jnp.float32),
            pltpu.VMEM((W, DH), jnp.float32),
            pltpu.VMEM_SHARED((G, DH), jnp.float32),
            pltpu.VMEM_SHARED((G, DH), jnp.float32),
        ],
        compiler_params=pltpu.CompilerParams(use_tc_tiling_on_sc=False),
        interpret=interpret,
    )
    return f(h0t, h1t, batch)


def _embed_tc(x, node_w, node_b, *, interpret=False):
    N, F = x.shape
    D = node_w.shape[1]
    B = _chunk(N, 10000)

    def body(x_ref, w_ref, b_ref, o_ref, o0_ref, o1_ref):
        h = jnp.dot(x_ref[...], w_ref[...],
                    preferred_element_type=jnp.float32) + b_ref[...]
        o_ref[...] = h
        o0_ref[...] = h[:, :DH]
        o1_ref[...] = h[:, DH:]

    return pl.pallas_call(
        body,
        grid=(N // B,),
        in_specs=[pl.BlockSpec((B, F), lambda i: (i, 0)),
                  pl.BlockSpec((F, D), lambda i: (0, 0)),
                  pl.BlockSpec((1, D), lambda i: (0, 0))],
        out_specs=(pl.BlockSpec((B, D), lambda i: (i, 0)),
                   pl.BlockSpec((B, DH), lambda i: (i, 0)),
                   pl.BlockSpec((B, DH), lambda i: (i, 0))),
        out_shape=(jax.ShapeDtypeStruct((N, D), jnp.float32),
                   jax.ShapeDtypeStruct((N, DH), jnp.float32),
                   jax.ShapeDtypeStruct((N, DH), jnp.float32)),
        interpret=interpret,
    )(x, node_w, node_b.reshape(1, D))


def _mlp_stats_tc(h, a0, a1, w1, b1, w2, b2, *, interpret=False):
    N, D = h.shape
    H = w1.shape[1]
    B = _chunk(N, 10000)

    def body(h_ref, a0_ref, a1_ref, w1_ref, b1_ref, w2_ref, b2_ref,
             z_ref, s_ref, q_ref):
        i = pl.program_id(0)
        xx = h_ref[...] + jnp.concatenate([a0_ref[...], a1_ref[...]], axis=1)
        t = jnp.maximum(jnp.dot(xx, w1_ref[...],
                                preferred_element_type=jnp.float32)
                        + b1_ref[...], 0.0)
        z = jnp.dot(t, w2_ref[...], preferred_element_type=jnp.float32) \
            + b2_ref[...]
        z_ref[...] = z

        @pl.when(i == 0)
        def _():
            s_ref[...] = jnp.zeros_like(s_ref)
            q_ref[...] = jnp.zeros_like(q_ref)
        s_ref[...] += jnp.sum(z, axis=0, keepdims=True)
        q_ref[...] += jnp.sum(z * z, axis=0, keepdims=True)

    return pl.pallas_call(
        body,
        grid=(N // B,),
        in_specs=[pl.BlockSpec((B, D), lambda i: (i, 0)),
                  pl.BlockSpec((B, DH), lambda i: (i, 0)),
                  pl.BlockSpec((B, DH), lambda i: (i, 0)),
                  pl.BlockSpec((D, H), lambda i: (0, 0)),
                  pl.BlockSpec((1, H), lambda i: (0, 0)),
                  pl.BlockSpec((H, D), lambda i: (0, 0)),
                  pl.BlockSpec((1, D), lambda i: (0, 0))],
        out_specs=(pl.BlockSpec((B, D), lambda i: (i, 0)),
                   pl.BlockSpec((1, D), lambda i: (0, 0)),
                   pl.BlockSpec((1, D), lambda i: (0, 0))),
        out_shape=(jax.ShapeDtypeStruct((N, D), jnp.float32),
                   jax.ShapeDtypeStruct((1, D), jnp.float32),
                   jax.ShapeDtypeStruct((1, D), jnp.float32)),
        interpret=interpret,
    )(h, a0, a1, w1, b1.reshape(1, H), w2, b2.reshape(1, D))


def _bn_relu_tc(z, zsum, zsq, g, bt, n_rows, *, interpret=False):
    N, D = z.shape
    B = _chunk(N, 10000)

    def body(z_ref, s_ref, q_ref, g_ref, b_ref, o_ref, o0_ref, o1_ref):
        inv_n = jnp.float32(1.0 / n_rows)
        mean = s_ref[...] * inv_n
        var = q_ref[...] * inv_n - mean * mean
        scale = g_ref[...] * lax.rsqrt(var + EPS_BN_)
        shift = b_ref[...] - mean * scale
        h = jnp.maximum(z_ref[...] * scale + shift, 0.0)
        o_ref[...] = h
        o0_ref[...] = h[:, :DH]
        o1_ref[...] = h[:, DH:]

    return pl.pallas_call(
        body,
        grid=(N // B,),
        in_specs=[pl.BlockSpec((B, D), lambda i: (i, 0)),
                  pl.BlockSpec((1, D), lambda i: (0, 0)),
                  pl.BlockSpec((1, D), lambda i: (0, 0)),
                  pl.BlockSpec((1, D), lambda i: (0, 0)),
                  pl.BlockSpec((1, D), lambda i: (0, 0))],
        out_specs=(pl.BlockSpec((B, D), lambda i: (i, 0)),
                   pl.BlockSpec((B, DH), lambda i: (i, 0)),
                   pl.BlockSpec((B, DH), lambda i: (i, 0))),
        out_shape=(jax.ShapeDtypeStruct((N, D), jnp.float32),
                   jax.ShapeDtypeStruct((N, DH), jnp.float32),
                   jax.ShapeDtypeStruct((N, DH), jnp.float32)),
        interpret=interpret,
    )(z, zsum, zsq, g.reshape(1, D), bt.reshape(1, D))


def _head_tc(p0, p1, cnt, l1w, l1b, l2w, l2b, *, interpret=False):
    G = p0.shape[0]
    D = 2 * DH
    H = l1w.shape[1]
    O = l2w.shape[1]

    def body(p0_ref, p1_ref, c_ref, w1_ref, b1_ref, w2_ref, b2_ref, o_ref):
        ssum = jnp.concatenate([p0_ref[...], p1_ref[...]], axis=1)
        c = jnp.maximum(c_ref[...][:, 0:1], 1.0)
        gx = ssum / c
        t = jnp.maximum(jnp.dot(gx, w1_ref[...],
                                preferred_element_type=jnp.float32)
                        + b1_ref[...], 0.0)
        o_ref[...] = jnp.dot(t, w2_ref[...],
                             preferred_element_type=jnp.float32) + b2_ref[...]

    return pl.pallas_call(
        body,
        out_shape=jax.ShapeDtypeStruct((G, O), jnp.float32),
        interpret=interpret,
    )(p0, p1, cnt, l1w, l1b.reshape(1, H), l2w, l2b.reshape(1, O))


def _forward(x, edge_index, edge_attr, batch,
             node_w, node_b, edge_w, edge_b,
             conv0_w1, conv0_b1, conv0_w2, conv0_b2, bn0_g, bn0_b,
             conv1_w1, conv1_b1, conv1_w2, conv1_b2, bn1_g, bn1_b,
             lin1_w, lin1_b, lin2_w, lin2_b, G, interpret=False):
    N = x.shape[0]
    src = edge_index[0]
    dst = edge_index[1]
    ew2 = edge_w.reshape(3, 2, DH).transpose(1, 0, 2).reshape(6 * DH)
    eb2 = edge_b.reshape(2 * DH)
    attr_flat = edge_attr.reshape(-1)

    h, h0t, h1t = _embed_tc(x, node_w, node_b, interpret=interpret)
    layers = [
        (conv0_w1, conv0_b1, conv0_w2, conv0_b2, bn0_g, bn0_b),
        (conv1_w1, conv1_b1, conv1_w2, conv1_b2, bn1_g, bn1_b),
    ]
    for (w1, b1, w2, b2, g, bt) in layers:
        a0, a1 = _edge_phase(h0t, h1t, src, dst, attr_flat, ew2, eb2,
                             interpret=interpret)
        z, zsum, zsq = _mlp_stats_tc(h, a0, a1, w1, b1, w2, b2,
                                     interpret=interpret)
        h, h0t, h1t = _bn_relu_tc(z, zsum, zsq, g, bt, N, interpret=interpret)

    p0, p1, cnt = _pool_phase(h0t, h1t, batch, G, interpret=interpret)
    return _head_tc(p0, p1, cnt, lin1_w, lin1_b, lin2_w, lin2_b,
                    interpret=interpret)


def kernel(x, edge_index, edge_attr, batch,
           node_w, node_b, edge_w, edge_b,
           conv0_w1, conv0_b1, conv0_w2, conv0_b2, bn0_g, bn0_b,
           conv1_w1, conv1_b1, conv1_w2, conv1_b2, bn1_g, bn1_b,
           lin1_w, lin1_b, lin2_w, lin2_b):
    return _forward(x, edge_index, edge_attr, batch,
                    node_w, node_b, edge_w, edge_b,
                    conv0_w1, conv0_b1, conv0_w2, conv0_b2, bn0_g, bn0_b,
                    conv1_w1, conv1_b1, conv1_w2, conv1_b2, bn1_g, bn1_b,
                    lin1_w, lin1_b, lin2_w, lin2_b, G=2000)
